# baseline (speedup 1.0000x reference)
import dataclasses

import jax
import jax.numpy as jnp
from jax.experimental import pallas as pl
from jax.experimental.pallas import tpu as pltpu
from jax.experimental.pallas import tpu_sc as plsc

B = 8192
D = 4096
H = 256
PROJ = 16
E = 64
K = 8

BM = 1024
SC_ROWS = 32
LANES = 16


def _logits_kernel(temp_ref, x_ref, w1_ref, b1_ref, w2_ref, b2_ref, en_ref,
                   out_ref):
    f32 = jnp.float32
    xb = x_ref[...].astype(jnp.bfloat16)
    h = jnp.dot(xb, w1_ref[...], preferred_element_type=f32)
    h = jnp.maximum(h + b1_ref[...], 0.0)
    proj = jnp.dot(h.astype(jnp.bfloat16), w2_ref[...],
                   preferred_element_type=f32)
    proj = proj + b2_ref[...]
    pn = proj * jax.lax.rsqrt(
        jnp.maximum(jnp.sum(proj * proj, axis=1, keepdims=True), 1e-12))
    pn = pn / temp_ref[0, 0]
    en = en_ref[...]
    en_n = en * jax.lax.rsqrt(
        jnp.maximum(jnp.sum(en * en, axis=1, keepdims=True), 1e-12))
    out_ref[...] = jnp.dot(pn.astype(jnp.bfloat16),
                           en_n.astype(jnp.bfloat16).T,
                           preferred_element_type=f32)


def _tc_logits(x, w1, b1r, w2, b2r, en, temp):
    rows = x.shape[0]
    const = lambda i: (0, 0)
    return pl.pallas_call(
        _logits_kernel,
        grid=(rows // BM,),
        in_specs=[
            pl.BlockSpec(memory_space=pltpu.SMEM),
            pl.BlockSpec((BM, D), lambda i: (i, 0)),
            pl.BlockSpec((D, H), const),
            pl.BlockSpec((1, H), const),
            pl.BlockSpec((H, PROJ), const),
            pl.BlockSpec((1, PROJ), const),
            pl.BlockSpec((E, PROJ), const),
        ],
        out_specs=pl.BlockSpec((BM, E), lambda i: (i, 0)),
        out_shape=jax.ShapeDtypeStruct((rows, E), jnp.float32),
        compiler_params=pltpu.CompilerParams(
            dimension_semantics=("parallel",)),
    )(temp, x, w1, b1r, w2, b2r, en)


def _sc_routing_body(in_vmem, out_vmem):
    neg = jnp.float32(-jnp.inf)
    ncol = E // LANES

    @pl.loop(0, SC_ROWS)
    def _(t):
        v = [in_vmem[t, pl.ds(c * LANES, LANES)] for c in range(ncol)]
        cur = list(v)
        mx = None
        for k in range(K):
            m01 = jnp.maximum(cur[0], cur[1])
            m23 = jnp.maximum(cur[2], cur[3])
            m = jnp.max(jnp.maximum(m01, m23))
            if k == 0:
                mx = m
            cur = [jnp.where(c == m, neg, c) for c in cur]
        e = [jnp.where(c == neg, jnp.exp(x - mx), 0.0)
             for c, x in zip(cur, v)]
        s = jnp.sum(e[0] + e[1] + e[2] + e[3])
        for c in range(ncol):
            out_vmem[t, pl.ds(c * LANES, LANES)] = e[c] / s


def _sc_routing(logits):
    rows = logits.shape[0]
    mesh = plsc.VectorSubcoreMesh(core_axis_name="core",
                                  subcore_axis_name="subcore")

    cp = pltpu.CompilerParams()
    if "needs_layout_passes" in pltpu.CompilerParams.__dataclass_fields__:
        cp = dataclasses.replace(cp, needs_layout_passes=False)

    @pl.kernel(out_type=jax.ShapeDtypeStruct((rows, E), jnp.float32),
               mesh=mesh, compiler_params=cp)
    def sc_kernel(in_hbm, out_hbm):
        pltpu.emit_pipeline(
            _sc_routing_body,
            grid=(rows // SC_ROWS,),
            in_specs=[pl.BlockSpec((SC_ROWS, E), lambda i: (i, 0))],
            out_specs=[pl.BlockSpec((SC_ROWS, E), lambda i: (i, 0))],
            core_axis_name=("core", "subcore"),
            dimension_semantics=(pltpu.PARALLEL,),
        )(in_hbm, out_hbm)

    return sc_kernel(logits)


@jax.jit
def kernel(x, W1, b1, W2, b2, expert_embedding, temperature):
    w1 = W1.astype(jnp.bfloat16)
    w2 = W2.astype(jnp.bfloat16)
    b1r = b1.reshape(1, H)
    b2r = b2.reshape(1, PROJ)
    temp = temperature.reshape(1, 1)
    nchunks = 4
    rows = B // nchunks
    gates = []
    for c in range(nchunks):
        xc = jax.lax.slice_in_dim(x, c * rows, (c + 1) * rows, axis=0)
        logits = _tc_logits(xc, w1, b1r, w2, b2r, expert_embedding, temp)
        gates.append(_sc_routing(logits))
    return jnp.concatenate(gates, axis=0)


# speedup vs baseline: 3.1622x; 3.1622x over previous
import jax
import jax.numpy as jnp
from jax.experimental import pallas as pl
from jax.experimental.pallas import tpu as pltpu

B = 8192
D = 4096
H = 256
PROJ = 16
E = 64
K = 8

BM = 1024


def _gating_kernel(temp_ref, x_ref, w1_ref, b1_ref, w2_ref, b2_ref, en_ref,
                   out_ref):
    f32 = jnp.float32
    xb = x_ref[...].astype(jnp.bfloat16)
    h = jnp.dot(xb, w1_ref[...], preferred_element_type=f32)
    h = jnp.maximum(h + b1_ref[...], 0.0)
    proj = jnp.dot(h.astype(jnp.bfloat16), w2_ref[...],
                   preferred_element_type=f32)
    proj = proj + b2_ref[...]
    pn = proj * jax.lax.rsqrt(
        jnp.maximum(jnp.sum(proj * proj, axis=1, keepdims=True), 1e-12))
    pn = pn / temp_ref[0, 0]
    en = en_ref[...]
    en_n = en * jax.lax.rsqrt(
        jnp.maximum(jnp.sum(en * en, axis=1, keepdims=True), 1e-12))
    logits_t = jax.lax.dot_general(
        en_n.astype(jnp.bfloat16), pn.astype(jnp.bfloat16),
        (((1,), (1,)), ((), ())), preferred_element_type=f32)

    neg = jnp.float32(-jnp.inf)
    cur = logits_t
    mx = None
    for k in range(K):
        m = jnp.max(cur, axis=0, keepdims=True)
        if k == 0:
            mx = m
        cur = jnp.where(cur == m, neg, cur)

    p = jnp.where(cur == neg, jnp.exp(logits_t - mx), 0.0)
    g = p / jnp.sum(p, axis=0, keepdims=True)
    out_ref[...] = g.T


@jax.jit
def kernel(x, W1, b1, W2, b2, expert_embedding, temperature):
    w1 = W1.astype(jnp.bfloat16)
    w2 = W2.astype(jnp.bfloat16)
    b1r = b1.reshape(1, H)
    b2r = b2.reshape(1, PROJ)
    temp = temperature.reshape(1, 1)

    grid = (B // BM,)
    const = lambda i: (0, 0)
    out = pl.pallas_call(
        _gating_kernel,
        grid=grid,
        in_specs=[
            pl.BlockSpec(memory_space=pltpu.SMEM),
            pl.BlockSpec((BM, D), lambda i: (i, 0)),
            pl.BlockSpec((D, H), const),
            pl.BlockSpec((1, H), const),
            pl.BlockSpec((H, PROJ), const),
            pl.BlockSpec((1, PROJ), const),
            pl.BlockSpec((E, PROJ), const),
        ],
        out_specs=pl.BlockSpec((BM, E), lambda i: (i, 0)),
        out_shape=jax.ShapeDtypeStruct((B, E), jnp.float32),
        compiler_params=pltpu.CompilerParams(
            dimension_semantics=("parallel",)),
    )(temp, x, w1, b1r, w2, b2r, expert_embedding)
    return out
